# baseline (speedup 1.0000x reference)
import dataclasses
import functools
import math

import jax
import jax.numpy as jnp
from jax import lax
from jax.experimental import pallas as pl
from jax.experimental.pallas import tpu as pltpu
from jax.experimental.pallas import tpu_sc as plsc

N = 10000
E = 320000
C = 128
H = 64
R = 50
RP = 64
CUTOFF = 5.0

NC = 2
NS = 16
LANES = 16
NW = NC * NS

ROWS = 2560
EP = ROWS * 128
RPW = ROWS // NW
NP = 10112
NPS = NP // NS

BE = 1024
BN = 1000


def _sc_geom_body(px_hbm, py_hbm, pz_hbm, src_hbm, dst_hbm,
                  d2_hbm,
                  px, py, pz, srcv, dstv, d2v):
    c = lax.axis_index("c")
    s = lax.axis_index("s")
    wid = c * NS + s
    base = wid * RPW

    pltpu.sync_copy(px_hbm, px)
    pltpu.sync_copy(py_hbm, py)
    pltpu.sync_copy(pz_hbm, pz)
    pltpu.sync_copy(src_hbm.at[pl.ds(base, RPW)], srcv)
    pltpu.sync_copy(dst_hbm.at[pl.ds(base, RPW)], dstv)

    @pl.loop(0, RPW)
    def _row(r):
        @pl.loop(0, 128 // LANES)
        def _vec(j):
            si = srcv[r, pl.ds(j * LANES, LANES)]
            di = dstv[r, pl.ds(j * LANES, LANES)]
            dx = plsc.load_gather(px, [si]) - plsc.load_gather(px, [di])
            dy = plsc.load_gather(py, [si]) - plsc.load_gather(py, [di])
            dz = plsc.load_gather(pz, [si]) - plsc.load_gather(pz, [di])
            d2v[r, pl.ds(j * LANES, LANES)] = dx * dx + dy * dy + dz * dz

    pltpu.sync_copy(d2v, d2_hbm.at[pl.ds(base, RPW)])


def _mul_chunk(xbuf, wbuf):
    @pl.loop(0, 128)
    def _row(i):
        for j in range(0, 128, LANES):
            xbuf[i, pl.ds(j, LANES)] = (
                xbuf[i, pl.ds(j, LANES)] * wbuf[i, pl.ds(j, LANES)])


def _sc_scatter_body(rpw, we_hbm, sd_hbm, x_hbm, z_hbm,
                     agg_hbm,
                     ia, ib, wb, xb0, xb1, agg_sh,
                     sw, sx0, sx1, sia, sib):
    c = lax.axis_index("c")
    s = lax.axis_index("s")
    wid = c * NS + s
    base = wid * rpw

    pltpu.sync_copy(z_hbm, agg_sh.at[pl.ds(s * NPS, NPS)])
    plsc.subcore_barrier()

    def start_idx(r, ibuf, sem):
        pltpu.async_copy(sd_hbm.at[pl.ds(base + r, 1)], ibuf, sem)

    def wait_idx(r, ibuf, sem):
        pltpu.make_async_copy(sd_hbm.at[pl.ds(base + r, 1)], ibuf, sem).wait()

    def start_w(r):
        pltpu.async_copy(we_hbm.at[pl.ds((base + r) * 128, 128)], wb, sw)

    def wait_w(r):
        pltpu.make_async_copy(
            we_hbm.at[pl.ds((base + r) * 128, 128)], wb, sw).wait()

    def start_x(r, ibuf, xb, sx):
        pltpu.async_copy(x_hbm.at[ibuf.at[0, 0, pl.ds(0, 64)]],
                         xb.at[pl.ds(0, 64)], sx)
        pltpu.async_copy(x_hbm.at[ibuf.at[0, 0, pl.ds(64, 64)]],
                         xb.at[pl.ds(64, 64)], sx)

    def wait_x(r, ibuf, xb, sx):
        pltpu.make_async_copy(x_hbm.at[ibuf.at[0, 0, pl.ds(0, 64)]],
                              xb.at[pl.ds(0, 64)], sx).wait()
        pltpu.make_async_copy(x_hbm.at[ibuf.at[0, 0, pl.ds(64, 64)]],
                              xb.at[pl.ds(64, 64)], sx).wait()

    start_idx(0, ia, sia)
    wait_idx(0, ia, sia)
    start_idx(1, ib, sib)
    start_x(0, ia, xb0, sx0)
    start_w(0)

    @pl.loop(0, rpw, step=2)
    def _pair(r):
        wait_idx(r + 1, ib, sib)
        start_x(r + 1, ib, xb1, sx1)
        wait_w(r)
        wait_x(r, ia, xb0, sx0)
        _mul_chunk(xb0, wb)
        start_w(r + 1)
        pltpu.sync_copy(xb0, agg_sh.at[ia.at[0, 1]], add=True)

        @pl.when(r + 2 < rpw)
        def _():
            start_idx(r + 2, ia, sia)

        wait_w(r + 1)
        wait_x(r + 1, ib, xb1, sx1)
        _mul_chunk(xb1, wb)
        pltpu.sync_copy(xb1, agg_sh.at[ib.at[0, 1]], add=True)

        @pl.when(r + 2 < rpw)
        def _():
            wait_idx(r + 2, ia, sia)
            start_x(r + 2, ia, xb0, sx0)
            start_w(r + 2)
            start_idx(r + 3, ib, sib)

    plsc.subcore_barrier()
    pltpu.sync_copy(agg_sh.at[pl.ds(s * NPS, NPS)],
                    agg_hbm.at[pl.ds(c * NP + s * NPS, NPS)])


LOG2E = 1.4426950408889634

_EXP2_C = (0.009618129107628477, 0.05550410866482158,
           0.2402265069591007, 0.6931471805599453, 1.0)
_RND = 12582912.0
_RND_BITS = 1262485504


def _fast_exp2(x):
    x = jnp.maximum(x, -126.0)
    t = x + _RND
    n_f = t - _RND
    f = x - n_f
    p = _EXP2_C[0]
    for c in _EXP2_C[1:]:
        p = p * f + c
    bits = jax.lax.bitcast_convert_type(t, jnp.int32)
    scale = jax.lax.bitcast_convert_type(
        jnp.left_shift(bits - (_RND_BITS - 127), 23), jnp.float32)
    return p * scale


HB = BE // 2


def _edge_mlp_body(d2_ref, means2_ref, betas2_ref,
                   W1p_ref, b1p_ref, W2p_ref, b2p_ref, we_ref):
    d2 = d2_ref[...]
    dist = jnp.sqrt(d2 + 1e-12)
    exp_dist = _fast_exp2(-LOG2E * dist)
    t = jnp.minimum(dist, CUTOFF) * (math.pi / CUTOFF) - (math.pi / 2)
    q = t * t
    sin_t = t * ((((q * 2.7557319223985893e-06 - 1.984126984126984e-04)
                   * q + 8.333333333333333e-03)
                  * q - 1.6666666666666666e-01) * q + 1.0)
    cut = 0.5 - 0.5 * sin_t
    cut = jnp.where(dist < CUTOFF, cut, 0.0)
    lane = lax.broadcasted_iota(jnp.int32, (1, 2 * RP), 1)
    lo = (lane < RP).astype(jnp.float32)
    hi = 1.0 - lo
    ed2 = (jnp.dot(exp_dist[:HB], lo, preferred_element_type=jnp.float32)
           + jnp.dot(exp_dist[HB:], hi, preferred_element_type=jnp.float32))
    cut2 = (jnp.dot(cut[:HB], lo, preferred_element_type=jnp.float32)
            + jnp.dot(cut[HB:], hi, preferred_element_type=jnp.float32))
    diff = ed2 - means2_ref[...]
    rbf = _fast_exp2(-betas2_ref[...] * diff * diff) * cut2
    h = jnp.dot(rbf, W1p_ref[...],
                preferred_element_type=jnp.float32) + b1p_ref[...]
    h = 0.5 * h * (jnp.tanh(0.5 * h) + 1.0)
    w = jnp.dot(h, W2p_ref[...],
                preferred_element_type=jnp.float32) + b2p_ref[...]
    we_ref[:HB, :] = w[:, :C]
    we_ref[HB:, :] = w[:, C:]


def _node_update_body(x_ref, a0_ref, a1_ref, Ws_ref, Wu_ref, bu_ref, out_ref):
    x = x_ref[...]
    agg = a0_ref[...] + a1_ref[...]
    t = jnp.dot(agg, Wu_ref[...],
                preferred_element_type=jnp.float32) + bu_ref[...]
    t = 0.5 * t * (jnp.tanh(0.5 * t) + 1.0)
    out_ref[...] = x + jnp.dot(x, Ws_ref[...],
                               preferred_element_type=jnp.float32) + t


def kernel(x, pos, edge_index, means, betas, W1, b1, W2, b2,
           W_self, W_upd, b_upd):
    src = edge_index[0]
    dst = edge_index[1]
    pad = EP - E
    src_p = jnp.concatenate(
        [src, jnp.zeros((pad,), jnp.int32)]).reshape(ROWS, 128)
    dst_p = jnp.concatenate(
        [dst, jnp.zeros((pad,), jnp.int32)]).reshape(ROWS, 128)
    posT = pos.T.reshape(3, N)
    px_a, py_a, pz_a = posT[0], posT[1], posT[2]

    mesh = plsc.VectorSubcoreMesh(
        core_axis_name="c", subcore_axis_name="s",
        num_cores=NC, num_subcores=NS)
    sc_params = pltpu.CompilerParams()
    if "needs_layout_passes" in pltpu.CompilerParams.__dataclass_fields__:
        sc_params = dataclasses.replace(sc_params, needs_layout_passes=False)

    sc_geom = pl.kernel(
        _sc_geom_body,
        out_type=jax.ShapeDtypeStruct((ROWS, 128), jnp.float32),
        mesh=mesh,
        scratch_types=[
            pltpu.VMEM((N,), jnp.float32),
            pltpu.VMEM((N,), jnp.float32),
            pltpu.VMEM((N,), jnp.float32),
            pltpu.VMEM((RPW, 128), jnp.int32),
            pltpu.VMEM((RPW, 128), jnp.int32),
            pltpu.VMEM((RPW, 128), jnp.float32),
        ],
        compiler_params=sc_params,
    )
    d2_2d = sc_geom(px_a, py_a, pz_a, src_p, dst_p)

    d2_col = d2_2d.reshape(EP, 1)
    means_p = jnp.zeros((1, RP), jnp.float32).at[0, :R].set(means)
    betas_p = jnp.zeros((1, RP), jnp.float32).at[0, :R].set(betas * LOG2E)
    W1_p = jnp.zeros((RP, H), jnp.float32).at[:R].set(W1)
    means2 = jnp.concatenate([means_p, means_p], axis=1)
    betas2 = jnp.concatenate([betas_p, betas_p], axis=1)
    W1_pair = jnp.zeros((2 * RP, 2 * H), jnp.float32)
    W1_pair = W1_pair.at[:RP, :H].set(W1_p).at[RP:, H:].set(W1_p)
    b1_pair = jnp.concatenate([b1, b1]).reshape(1, 2 * H)
    W2_pair = jnp.zeros((2 * H, 2 * C), jnp.float32)
    W2_pair = W2_pair.at[:H, :C].set(W2).at[H:, C:].set(W2)
    b2_pair = jnp.concatenate([b2, b2]).reshape(1, 2 * C)
    def edge_mlp(d2_half, n_edges):
        return pl.pallas_call(
            _edge_mlp_body,
            grid=(n_edges // BE,),
            in_specs=[
                pl.BlockSpec((BE, 1), lambda i: (i, 0)),
                pl.BlockSpec((1, 2 * RP), lambda i: (0, 0)),
                pl.BlockSpec((1, 2 * RP), lambda i: (0, 0)),
                pl.BlockSpec((2 * RP, 2 * H), lambda i: (0, 0)),
                pl.BlockSpec((1, 2 * H), lambda i: (0, 0)),
                pl.BlockSpec((2 * H, 2 * C), lambda i: (0, 0)),
                pl.BlockSpec((1, 2 * C), lambda i: (0, 0)),
            ],
            out_specs=pl.BlockSpec((BE, C), lambda i: (i, 0)),
            out_shape=jax.ShapeDtypeStruct((n_edges, C), jnp.float32),
        )(d2_half, means2, betas2, W1_pair, b1_pair, W2_pair, b2_pair)

    RW1 = 3 * ROWS // 8
    EP1 = RW1 * 128
    sd = jnp.stack([src_p, dst_p], axis=1)

    def make_scatter(rpw):
        return pl.kernel(
            functools.partial(_sc_scatter_body, rpw),
            out_type=jax.ShapeDtypeStruct((NC * NP, C), jnp.float32),
            mesh=mesh,
            scratch_types=[
                pltpu.VMEM((1, 2, 128), jnp.int32),
                pltpu.VMEM((1, 2, 128), jnp.int32),
                pltpu.VMEM((128, C), jnp.float32),
                pltpu.VMEM((128, C), jnp.float32),
                pltpu.VMEM((128, C), jnp.float32),
                pltpu.VMEM_SHARED((NP, C), jnp.float32),
                pltpu.SemaphoreType.DMA,
                pltpu.SemaphoreType.DMA,
                pltpu.SemaphoreType.DMA,
                pltpu.SemaphoreType.DMA,
                pltpu.SemaphoreType.DMA,
            ],
            compiler_params=sc_params,
        )
    zeros_tile = jnp.zeros((NPS, C), jnp.float32)
    we1 = edge_mlp(d2_col[:EP1], EP1)
    we2 = edge_mlp(d2_col[EP1:], EP - EP1)
    we2 = we2.at[E - EP1:].set(0.0)
    agg2a = make_scatter(RW1 // NW)(we1, sd[:RW1], x, zeros_tile)
    agg2b = make_scatter((ROWS - RW1) // NW)(we2, sd[RW1:], x, zeros_tile)
    agg2 = agg2a + agg2b

    out = pl.pallas_call(
        _node_update_body,
        grid=(N // BN,),
        in_specs=[
            pl.BlockSpec((BN, C), lambda i: (i, 0)),
            pl.BlockSpec((BN, C), lambda i: (i, 0)),
            pl.BlockSpec((BN, C), lambda i: (i, 0)),
            pl.BlockSpec((C, C), lambda i: (0, 0)),
            pl.BlockSpec((C, C), lambda i: (0, 0)),
            pl.BlockSpec((1, C), lambda i: (0, 0)),
        ],
        out_specs=pl.BlockSpec((BN, C), lambda i: (i, 0)),
        out_shape=jax.ShapeDtypeStruct((N, C), jnp.float32),
    )(x, agg2[:N], agg2[NP:NP + N], W_self, W_upd, b_upd.reshape(1, C))
    return out


# speedup vs baseline: 1.0286x; 1.0286x over previous
import dataclasses
import functools
import math

import jax
import jax.numpy as jnp
from jax import lax
from jax.experimental import pallas as pl
from jax.experimental.pallas import tpu as pltpu
from jax.experimental.pallas import tpu_sc as plsc

N = 10000
E = 320000
C = 128
H = 64
R = 50
RP = 64
CUTOFF = 5.0

NC = 2
NS = 16
LANES = 16
NW = NC * NS

ROWS = 2560
EP = ROWS * 128
RPW = ROWS // NW
NP = 10112
NPS = NP // NS

BE = 1024
BN = 1000


def _sc_geom_body(px_hbm, py_hbm, pz_hbm, src_hbm, dst_hbm,
                  d2_hbm,
                  px, py, pz, srcv, dstv, d2v):
    c = lax.axis_index("c")
    s = lax.axis_index("s")
    wid = c * NS + s
    base = wid * RPW

    pltpu.sync_copy(px_hbm, px)
    pltpu.sync_copy(py_hbm, py)
    pltpu.sync_copy(pz_hbm, pz)
    pltpu.sync_copy(src_hbm.at[pl.ds(base, RPW)], srcv)
    pltpu.sync_copy(dst_hbm.at[pl.ds(base, RPW)], dstv)

    @pl.loop(0, RPW)
    def _row(r):
        @pl.loop(0, 128 // LANES)
        def _vec(j):
            si = srcv[r, pl.ds(j * LANES, LANES)]
            di = dstv[r, pl.ds(j * LANES, LANES)]
            dx = plsc.load_gather(px, [si]) - plsc.load_gather(px, [di])
            dy = plsc.load_gather(py, [si]) - plsc.load_gather(py, [di])
            dz = plsc.load_gather(pz, [si]) - plsc.load_gather(pz, [di])
            d2v[r, pl.ds(j * LANES, LANES)] = dx * dx + dy * dy + dz * dz

    pltpu.sync_copy(d2v, d2_hbm.at[pl.ds(base, RPW)])


def _mul_chunk(xbuf, wbuf):
    @pl.loop(0, 128)
    def _row(i):
        for j in range(0, 128, LANES):
            xbuf[i, pl.ds(j, LANES)] = (
                xbuf[i, pl.ds(j, LANES)] * wbuf[i, pl.ds(j, LANES)])


def _sc_scatter_body(rpw, we_hbm, sd_hbm, x_hbm, z_hbm,
                     agg_hbm,
                     ia, ib, wb, xb0, xb1, agg_sh,
                     sw, sx0, sx1, sia, sib):
    c = lax.axis_index("c")
    s = lax.axis_index("s")
    wid = c * NS + s
    base = wid * rpw

    pltpu.sync_copy(z_hbm, agg_sh.at[pl.ds(s * NPS, NPS)])
    plsc.subcore_barrier()

    def start_idx(r, ibuf, sem):
        pltpu.async_copy(sd_hbm.at[pl.ds(base + r, 1)], ibuf, sem)

    def wait_idx(r, ibuf, sem):
        pltpu.make_async_copy(sd_hbm.at[pl.ds(base + r, 1)], ibuf, sem).wait()

    def start_w(r):
        pltpu.async_copy(we_hbm.at[pl.ds((base + r) * 128, 128)], wb, sw)

    def wait_w(r):
        pltpu.make_async_copy(
            we_hbm.at[pl.ds((base + r) * 128, 128)], wb, sw).wait()

    def start_x(r, ibuf, xb, sx):
        pltpu.async_copy(x_hbm.at[ibuf.at[0, 0, pl.ds(0, 64)]],
                         xb.at[pl.ds(0, 64)], sx)
        pltpu.async_copy(x_hbm.at[ibuf.at[0, 0, pl.ds(64, 64)]],
                         xb.at[pl.ds(64, 64)], sx)

    def wait_x(r, ibuf, xb, sx):
        pltpu.make_async_copy(x_hbm.at[ibuf.at[0, 0, pl.ds(0, 64)]],
                              xb.at[pl.ds(0, 64)], sx).wait()
        pltpu.make_async_copy(x_hbm.at[ibuf.at[0, 0, pl.ds(64, 64)]],
                              xb.at[pl.ds(64, 64)], sx).wait()

    start_idx(0, ia, sia)
    wait_idx(0, ia, sia)
    start_idx(1, ib, sib)
    start_x(0, ia, xb0, sx0)
    start_w(0)

    @pl.loop(0, rpw, step=2)
    def _pair(r):
        wait_idx(r + 1, ib, sib)
        start_x(r + 1, ib, xb1, sx1)
        wait_w(r)
        wait_x(r, ia, xb0, sx0)
        _mul_chunk(xb0, wb)
        start_w(r + 1)
        pltpu.sync_copy(xb0, agg_sh.at[ia.at[0, 1]], add=True)

        @pl.when(r + 2 < rpw)
        def _():
            start_idx(r + 2, ia, sia)

        wait_w(r + 1)
        wait_x(r + 1, ib, xb1, sx1)
        _mul_chunk(xb1, wb)
        pltpu.sync_copy(xb1, agg_sh.at[ib.at[0, 1]], add=True)

        @pl.when(r + 2 < rpw)
        def _():
            wait_idx(r + 2, ia, sia)
            start_x(r + 2, ia, xb0, sx0)
            start_w(r + 2)
            start_idx(r + 3, ib, sib)

    plsc.subcore_barrier()
    pltpu.sync_copy(agg_sh.at[pl.ds(s * NPS, NPS)],
                    agg_hbm.at[pl.ds(c * NP + s * NPS, NPS)])


LOG2E = 1.4426950408889634

_EXP2_C = (0.009618129107628477, 0.05550410866482158,
           0.2402265069591007, 0.6931471805599453, 1.0)
_RND = 12582912.0
_RND_BITS = 1262485504


def _fast_exp2(x):
    x = jnp.maximum(x, -126.0)
    t = x + _RND
    n_f = t - _RND
    f = x - n_f
    p = _EXP2_C[0]
    for c in _EXP2_C[1:]:
        p = p * f + c
    bits = jax.lax.bitcast_convert_type(t, jnp.int32)
    scale = jax.lax.bitcast_convert_type(
        jnp.left_shift(bits - (_RND_BITS - 127), 23), jnp.float32)
    return p * scale


HB = BE // 2


def _edge_mlp_body(d2_ref, means2_ref, betas2_ref,
                   W1p_ref, b1p_ref, W2p_ref, b2p_ref, we_ref):
    d2 = d2_ref[...]
    dist = jnp.sqrt(d2 + 1e-12)
    exp_dist = _fast_exp2(-LOG2E * dist)
    t = jnp.minimum(dist, CUTOFF) * (math.pi / CUTOFF) - (math.pi / 2)
    q = t * t
    sin_t = t * ((((q * 2.7557319223985893e-06 - 1.984126984126984e-04)
                   * q + 8.333333333333333e-03)
                  * q - 1.6666666666666666e-01) * q + 1.0)
    cut = 0.5 - 0.5 * sin_t
    cut = jnp.where(dist < CUTOFF, cut, 0.0)
    lane = lax.broadcasted_iota(jnp.int32, (1, 2 * RP), 1)
    lo = (lane < RP).astype(jnp.float32)
    hi = 1.0 - lo
    ed2 = (jnp.dot(exp_dist[:HB], lo, preferred_element_type=jnp.float32)
           + jnp.dot(exp_dist[HB:], hi, preferred_element_type=jnp.float32))
    cut2 = (jnp.dot(cut[:HB], lo, preferred_element_type=jnp.float32)
            + jnp.dot(cut[HB:], hi, preferred_element_type=jnp.float32))
    diff = ed2 - means2_ref[...]
    rbf = _fast_exp2(-betas2_ref[...] * diff * diff) * cut2
    h = jnp.dot(rbf, W1p_ref[...],
                preferred_element_type=jnp.float32) + b1p_ref[...]
    h = 0.5 * h * (jnp.tanh(0.5 * h) + 1.0)
    w = jnp.dot(h, W2p_ref[...],
                preferred_element_type=jnp.float32) + b2p_ref[...]
    we_ref[:HB, :] = w[:, :C]
    we_ref[HB:, :] = w[:, C:]


def _node_update_body(x_ref, a0_ref, a1_ref, Ws_ref, Wu_ref, bu_ref, out_ref):
    x = x_ref[...]
    agg = a0_ref[...] + a1_ref[...]
    t = jnp.dot(agg, Wu_ref[...],
                preferred_element_type=jnp.float32) + bu_ref[...]
    t = 0.5 * t * (jnp.tanh(0.5 * t) + 1.0)
    out_ref[...] = x + jnp.dot(x, Ws_ref[...],
                               preferred_element_type=jnp.float32) + t


def kernel(x, pos, edge_index, means, betas, W1, b1, W2, b2,
           W_self, W_upd, b_upd):
    src = edge_index[0]
    dst = edge_index[1]
    pad = EP - E
    src_p = jnp.concatenate(
        [src, jnp.zeros((pad,), jnp.int32)]).reshape(ROWS, 128)
    dst_p = jnp.concatenate(
        [dst, jnp.zeros((pad,), jnp.int32)]).reshape(ROWS, 128)
    posT = pos.T.reshape(3, N)
    px_a, py_a, pz_a = posT[0], posT[1], posT[2]

    mesh = plsc.VectorSubcoreMesh(
        core_axis_name="c", subcore_axis_name="s",
        num_cores=NC, num_subcores=NS)
    sc_params = pltpu.CompilerParams()
    if "needs_layout_passes" in pltpu.CompilerParams.__dataclass_fields__:
        sc_params = dataclasses.replace(sc_params, needs_layout_passes=False)

    sc_geom = pl.kernel(
        _sc_geom_body,
        out_type=jax.ShapeDtypeStruct((ROWS, 128), jnp.float32),
        mesh=mesh,
        scratch_types=[
            pltpu.VMEM((N,), jnp.float32),
            pltpu.VMEM((N,), jnp.float32),
            pltpu.VMEM((N,), jnp.float32),
            pltpu.VMEM((RPW, 128), jnp.int32),
            pltpu.VMEM((RPW, 128), jnp.int32),
            pltpu.VMEM((RPW, 128), jnp.float32),
        ],
        compiler_params=sc_params,
    )
    d2_2d = sc_geom(px_a, py_a, pz_a, src_p, dst_p)

    d2_col = d2_2d.reshape(EP, 1)
    means_p = jnp.zeros((1, RP), jnp.float32).at[0, :R].set(means)
    betas_p = jnp.zeros((1, RP), jnp.float32).at[0, :R].set(betas * LOG2E)
    W1_p = jnp.zeros((RP, H), jnp.float32).at[:R].set(W1)
    means2 = jnp.concatenate([means_p, means_p], axis=1)
    betas2 = jnp.concatenate([betas_p, betas_p], axis=1)
    W1_pair = jnp.zeros((2 * RP, 2 * H), jnp.float32)
    W1_pair = W1_pair.at[:RP, :H].set(W1_p).at[RP:, H:].set(W1_p)
    b1_pair = jnp.concatenate([b1, b1]).reshape(1, 2 * H)
    W2_pair = jnp.zeros((2 * H, 2 * C), jnp.float32)
    W2_pair = W2_pair.at[:H, :C].set(W2).at[H:, C:].set(W2)
    b2_pair = jnp.concatenate([b2, b2]).reshape(1, 2 * C)
    def edge_mlp(d2_half, n_edges):
        return pl.pallas_call(
            _edge_mlp_body,
            grid=(n_edges // BE,),
            in_specs=[
                pl.BlockSpec((BE, 1), lambda i: (i, 0)),
                pl.BlockSpec((1, 2 * RP), lambda i: (0, 0)),
                pl.BlockSpec((1, 2 * RP), lambda i: (0, 0)),
                pl.BlockSpec((2 * RP, 2 * H), lambda i: (0, 0)),
                pl.BlockSpec((1, 2 * H), lambda i: (0, 0)),
                pl.BlockSpec((2 * H, 2 * C), lambda i: (0, 0)),
                pl.BlockSpec((1, 2 * C), lambda i: (0, 0)),
            ],
            out_specs=pl.BlockSpec((BE, C), lambda i: (i, 0)),
            out_shape=jax.ShapeDtypeStruct((n_edges, C), jnp.float32),
        )(d2_half, means2, betas2, W1_pair, b1_pair, W2_pair, b2_pair)

    RW1 = ROWS // 2
    EP1 = RW1 * 128
    sd = jnp.stack([src_p, dst_p], axis=1)

    def make_scatter(rpw):
        return pl.kernel(
            functools.partial(_sc_scatter_body, rpw),
            out_type=jax.ShapeDtypeStruct((NC * NP, C), jnp.float32),
            mesh=mesh,
            scratch_types=[
                pltpu.VMEM((1, 2, 128), jnp.int32),
                pltpu.VMEM((1, 2, 128), jnp.int32),
                pltpu.VMEM((128, C), jnp.float32),
                pltpu.VMEM((128, C), jnp.float32),
                pltpu.VMEM((128, C), jnp.float32),
                pltpu.VMEM_SHARED((NP, C), jnp.float32),
                pltpu.SemaphoreType.DMA,
                pltpu.SemaphoreType.DMA,
                pltpu.SemaphoreType.DMA,
                pltpu.SemaphoreType.DMA,
                pltpu.SemaphoreType.DMA,
            ],
            compiler_params=sc_params,
        )
    zeros_tile = jnp.zeros((NPS, C), jnp.float32)
    we1 = edge_mlp(d2_col[:EP1], EP1)
    we2 = edge_mlp(d2_col[EP1:], EP - EP1)
    we2 = we2.at[E - EP1:].set(0.0)
    agg2a = make_scatter(RW1 // NW)(we1, sd[:RW1], x, zeros_tile)
    agg2b = make_scatter((ROWS - RW1) // NW)(we2, sd[RW1:], x, zeros_tile)
    agg2 = agg2a + agg2b

    out = pl.pallas_call(
        _node_update_body,
        grid=(N // BN,),
        in_specs=[
            pl.BlockSpec((BN, C), lambda i: (i, 0)),
            pl.BlockSpec((BN, C), lambda i: (i, 0)),
            pl.BlockSpec((BN, C), lambda i: (i, 0)),
            pl.BlockSpec((C, C), lambda i: (0, 0)),
            pl.BlockSpec((C, C), lambda i: (0, 0)),
            pl.BlockSpec((1, C), lambda i: (0, 0)),
        ],
        out_specs=pl.BlockSpec((BN, C), lambda i: (i, 0)),
        out_shape=jax.ShapeDtypeStruct((N, C), jnp.float32),
    )(x, agg2[:N], agg2[NP:NP + N], W_self, W_upd, b_upd.reshape(1, C))
    return out
